# baseline (speedup 1.0000x reference)
import functools

import jax
import jax.numpy as jnp
from jax import lax
from jax.experimental import pallas as pl
from jax.experimental.pallas import tpu as pltpu
from jax.experimental.pallas import tpu_sc as plsc

_BATCH = 16384
_HIST = 50
_NDIM = 32
_NSTIM = 1000000

_NC = 2
_NS = 16
_NW = _NC * _NS

_NTILE = _NSTIM // 128
_TAIL0 = _NTILE * 128
_TAILN = _NSTIM - _TAIL0
_K1_PER_W = -(-_NTILE // _NW)
_K1_PAIRS = (_K1_PER_W + 1) // 2

_TRIALS_PER_W = _BATCH // _NW
_C = 16
_NCHUNK = _TRIALS_PER_W // _C
_IDX_PER_CHUNK = _C * _HIST
_G = 80
_NG = _IDX_PER_CHUNK // _G


def _build_k1():
    mesh = plsc.VectorSubcoreMesh(core_axis_name="c", subcore_axis_name="s")

    @functools.partial(
        pl.kernel,
        mesh=mesh,
        out_type=jax.ShapeDtypeStruct((_NSTIM * _NDIM,), jnp.float32),
        compiler_params=pltpu.CompilerParams(needs_layout_passes=False),
        scratch_types=[
            pltpu.VMEM((4, 8, 128), jnp.float32),
            pltpu.VMEM((4, 8, 128), jnp.float32),
            pltpu.VMEM((128 * _NDIM,), jnp.float32),
            pltpu.VMEM((128 * _NDIM,), jnp.float32),
            pltpu.VMEM((_TAILN * _NDIM,), jnp.float32),
            pltpu.SemaphoreType.DMA,
            pltpu.SemaphoreType.DMA,
            pltpu.SemaphoreType.DMA,
            pltpu.SemaphoreType.DMA,
        ],
    )
    def k1(zt_hbm, ztail_hbm, zlin_hbm, va, vb, ta, tb, tailv,
           sga, sgb, soa, sob):
        wid = lax.axis_index("s") * _NC + lax.axis_index("c")

        iota_d = jnp.arange(16, dtype=jnp.int32) * _NDIM

        def start_read(k, v, sg):
            c0 = k * 128
            for j in range(4):
                pltpu.async_copy(
                    zt_hbm.at[pl.ds(j * 8, 8), pl.ds(c0, 128)], v.at[j], sg
                )

        def wait_read(v, sg):
            for j in range(4):
                pltpu.make_async_copy(
                    zt_hbm.at[pl.ds(0, 8), pl.ds(0, 128)], v.at[j], sg
                ).wait()

        def transpose_tile(v, t):
            for j in range(4):
                for s in range(8):
                    d = j * 8 + s
                    for cc in range(8):
                        vec = v[j, s, pl.ds(cc * 16, 16)]
                        plsc.store_scatter(
                            t, [iota_d + (cc * 16 * _NDIM + d)], vec
                        )

        def start_write(k, t, so):
            pltpu.async_copy(
                t, zlin_hbm.at[pl.ds(k * (128 * _NDIM), 128 * _NDIM)], so
            )

        def wait_write(t, so):
            pltpu.make_async_copy(
                zlin_hbm.at[pl.ds(0, 128 * _NDIM)], t, so
            ).wait()

        def chunk_id(p, half):
            return wid + _NW * (2 * p + half)

        @pl.when(chunk_id(0, 0) < _NTILE)
        def _():
            start_read(chunk_id(0, 0), va, sga)

        def pair(p, carry):
            ka = chunk_id(p, 0)
            kb = chunk_id(p, 1)

            @pl.when(kb < _NTILE)
            def _():
                start_read(kb, vb, sgb)

            @pl.when(ka < _NTILE)
            def _():
                wait_read(va, sga)

                @pl.when(p > 0)
                def _():
                    wait_write(ta, soa)

                transpose_tile(va, ta)
                start_write(ka, ta, soa)

            @pl.when(chunk_id(p + 1, 0) < _NTILE)
            def _():
                start_read(chunk_id(p + 1, 0), va, sga)

            @pl.when(kb < _NTILE)
            def _():
                wait_read(vb, sgb)

                @pl.when(p > 0)
                def _():
                    wait_write(tb, sob)

                transpose_tile(vb, tb)
                start_write(kb, tb, sob)

            return carry

        lax.fori_loop(0, _K1_PAIRS, pair, 0)
        wait_write(ta, soa)
        wait_write(tb, sob)

        @pl.when(wid == 0)
        def _():
            pltpu.sync_copy(ztail_hbm, tailv)
            pltpu.sync_copy(
                tailv, zlin_hbm.at[pl.ds(_TAIL0 * _NDIM, _TAILN * _NDIM)]
            )

    return k1


def _build_k2():
    mesh = plsc.VectorSubcoreMesh(core_axis_name="c", subcore_axis_name="s")

    @functools.partial(
        pl.kernel,
        mesh=mesh,
        out_type=jax.ShapeDtypeStruct((_HIST, _NDIM, _BATCH), jnp.float32),
        compiler_params=pltpu.CompilerParams(
            needs_layout_passes=False, use_tc_tiling_on_sc=False
        ),
        scratch_types=[
            pltpu.VMEM((_IDX_PER_CHUNK,), jnp.int32),
            pltpu.VMEM((_IDX_PER_CHUNK,), jnp.int32),
            pltpu.VMEM((_IDX_PER_CHUNK, _NDIM), jnp.float32),
            pltpu.VMEM((_IDX_PER_CHUNK, _NDIM), jnp.float32),
            pltpu.VMEM((_HIST, _NDIM, _C), jnp.float32),
            pltpu.VMEM((_HIST, _NDIM, _C), jnp.float32),
            pltpu.SemaphoreType.DMA,
            pltpu.SemaphoreType.DMA,
            pltpu.SemaphoreType.DMA,
            pltpu.SemaphoreType.DMA,
        ],
    )
    def k2(idx_hbm, z_hbm, out_hbm, idx0, idx1, rows0, rows1,
           ob0, ob1, sg0, sg1, so0, so1):
        wid = lax.axis_index("s") * _NC + lax.axis_index("c")
        idx_base = wid * (_TRIALS_PER_W * _HIST)
        b_base = wid * _TRIALS_PER_W

        iota_b = jnp.arange(16, dtype=jnp.int32) * _HIST

        def start(g, idx_v, rows_v, sg):
            i0 = idx_base + g * _IDX_PER_CHUNK
            pltpu.sync_copy(idx_hbm.at[pl.ds(i0, _IDX_PER_CHUNK)], idx_v)
            for j in range(_NG):
                pltpu.async_copy(
                    z_hbm.at[idx_v.at[pl.ds(j * _G, _G)]],
                    rows_v.at[pl.ds(j * _G, _G)],
                    sg,
                )

        def wait_gather(rows_v, sg):
            pltpu.make_async_copy(
                z_hbm.at[pl.ds(0, _IDX_PER_CHUNK)], rows_v, sg
            ).wait()

        def relayout(rows_v, out_v):
            def hbody(h, carry):
                rowpat = iota_b + h
                for d in range(_NDIM):
                    vec = plsc.load_gather(
                        rows_v, [rowpat, jnp.full((16,), d, jnp.int32)]
                    )
                    out_v[h, d] = vec
                return carry

            lax.fori_loop(0, _HIST, hbody, 0)

        def start_out(g, out_v, so):
            pltpu.async_copy(
                out_v,
                out_hbm.at[:, :, pl.ds(b_base + g * _C, _C)],
                so,
            )

        def wait_out(out_v, so):
            pltpu.make_async_copy(
                out_hbm.at[:, :, pl.ds(0, _C)], out_v, so
            ).wait()

        start(0, idx0, rows0, sg0)

        def pair(p, carry):
            g0 = 2 * p
            g1 = g0 + 1
            start(g1, idx1, rows1, sg1)
            wait_gather(rows0, sg0)

            @pl.when(p > 0)
            def _():
                wait_out(ob0, so0)

            relayout(rows0, ob0)
            start_out(g0, ob0, so0)

            @pl.when(p < _NCHUNK // 2 - 1)
            def _():
                start(g0 + 2, idx0, rows0, sg0)

            wait_gather(rows1, sg1)

            @pl.when(p > 0)
            def _():
                wait_out(ob1, so1)

            relayout(rows1, ob1)
            start_out(g1, ob1, so1)
            return carry

        lax.fori_loop(0, _NCHUNK // 2, pair, 0)
        wait_out(ob0, so0)
        wait_out(ob1, so1)

    return k2


_K1 = _build_k1()
_K2 = _build_k2()


@jax.jit
def kernel(inputs, z):
    zt = jnp.transpose(z)
    ztail = lax.slice(z, (_TAIL0, 0), (_NSTIM, _NDIM)).reshape(-1)
    z_lin = _K1(zt, ztail).reshape(_NSTIM, _NDIM)
    idx = jnp.reshape(inputs, (_BATCH * _HIST,))
    out_t = _K2(idx, z_lin)
    return jnp.transpose(out_t, (2, 1, 0))


# speedup vs baseline: 1.6457x; 1.6457x over previous
import functools

import jax
import jax.numpy as jnp
from jax import lax
from jax.experimental import pallas as pl
from jax.experimental.pallas import tpu as pltpu
from jax.experimental.pallas import tpu_sc as plsc

_BATCH = 16384
_HIST = 50
_NDIM = 32
_NSTIM = 1000000

_NC = 2
_NS = 16
_NW = _NC * _NS

_NTILE = _NSTIM // 128
_TAIL0 = _NTILE * 128
_TAILN = _NSTIM - _TAIL0
_K1_PER_W = -(-_NTILE // _NW)
_K1_PAIRS = (_K1_PER_W + 1) // 2

_TRIALS_PER_W = _BATCH // _NW
_C = 16
_NCHUNK = _TRIALS_PER_W // _C
_IDX_PER_CHUNK = _C * _HIST
_G = 80
_NG = _IDX_PER_CHUNK // _G


def _build_k1():
    mesh = plsc.VectorSubcoreMesh(core_axis_name="c", subcore_axis_name="s")

    @functools.partial(
        pl.kernel,
        mesh=mesh,
        out_type=jax.ShapeDtypeStruct((_NSTIM * _NDIM,), jnp.float32),
        compiler_params=pltpu.CompilerParams(needs_layout_passes=False),
        scratch_types=[
            pltpu.VMEM((4, 8, 128), jnp.float32),
            pltpu.VMEM((4, 8, 128), jnp.float32),
            pltpu.VMEM((128 * (_NDIM + 1),), jnp.float32),
            pltpu.VMEM((128 * _NDIM,), jnp.float32),
            pltpu.VMEM((128 * _NDIM,), jnp.float32),
            pltpu.VMEM((_TAILN * _NDIM,), jnp.float32),
            pltpu.SemaphoreType.DMA,
            pltpu.SemaphoreType.DMA,
            pltpu.SemaphoreType.DMA,
            pltpu.SemaphoreType.DMA,
        ],
    )
    def k1(zt_hbm, ztail_hbm, zlin_hbm, va, vb, tpad, ta, tb, tailv,
           sga, sgb, soa, sob):
        wid = lax.axis_index("s") * _NC + lax.axis_index("c")

        iota_p = jnp.arange(16, dtype=jnp.int32) * (_NDIM + 1)

        def start_read(k, v, sg):
            c0 = k * 128
            for j in range(4):
                pltpu.async_copy(
                    zt_hbm.at[pl.ds(j * 8, 8), pl.ds(c0, 128)], v.at[j], sg
                )

        def wait_read(v, sg):
            for j in range(4):
                pltpu.make_async_copy(
                    zt_hbm.at[pl.ds(0, 8), pl.ds(0, 128)], v.at[j], sg
                ).wait()

        def transpose_tile(v, t):
            for j in range(4):
                for s in range(8):
                    d = j * 8 + s
                    for cc in range(8):
                        vec = v[j, s, pl.ds(cc * 16, 16)]
                        plsc.store_scatter(
                            tpad, [iota_p + (cc * 16 * (_NDIM + 1) + d)], vec
                        )
            for c in range(128):
                t[pl.ds(c * _NDIM, 16)] = tpad[pl.ds(c * (_NDIM + 1), 16)]
                t[pl.ds(c * _NDIM + 16, 16)] = tpad[
                    pl.ds(c * (_NDIM + 1) + 16, 16)
                ]

        def start_write(k, t, so):
            pltpu.async_copy(
                t, zlin_hbm.at[pl.ds(k * (128 * _NDIM), 128 * _NDIM)], so
            )

        def wait_write(t, so):
            pltpu.make_async_copy(
                zlin_hbm.at[pl.ds(0, 128 * _NDIM)], t, so
            ).wait()

        def chunk_id(p, half):
            return wid + _NW * (2 * p + half)

        @pl.when(chunk_id(0, 0) < _NTILE)
        def _():
            start_read(chunk_id(0, 0), va, sga)

        def pair(p, carry):
            ka = chunk_id(p, 0)
            kb = chunk_id(p, 1)

            @pl.when(kb < _NTILE)
            def _():
                start_read(kb, vb, sgb)

            @pl.when(ka < _NTILE)
            def _():
                wait_read(va, sga)

                @pl.when(p > 0)
                def _():
                    wait_write(ta, soa)

                transpose_tile(va, ta)
                start_write(ka, ta, soa)

            @pl.when(chunk_id(p + 1, 0) < _NTILE)
            def _():
                start_read(chunk_id(p + 1, 0), va, sga)

            @pl.when(kb < _NTILE)
            def _():
                wait_read(vb, sgb)

                @pl.when(p > 0)
                def _():
                    wait_write(tb, sob)

                transpose_tile(vb, tb)
                start_write(kb, tb, sob)

            return carry

        lax.fori_loop(0, _K1_PAIRS, pair, 0)
        wait_write(ta, soa)
        wait_write(tb, sob)

        @pl.when(wid == 0)
        def _():
            pltpu.sync_copy(ztail_hbm, tailv)
            pltpu.sync_copy(
                tailv, zlin_hbm.at[pl.ds(_TAIL0 * _NDIM, _TAILN * _NDIM)]
            )

    return k1


def _build_k2():
    mesh = plsc.VectorSubcoreMesh(core_axis_name="c", subcore_axis_name="s")

    @functools.partial(
        pl.kernel,
        mesh=mesh,
        out_type=jax.ShapeDtypeStruct((_HIST, _NDIM, _BATCH), jnp.float32),
        compiler_params=pltpu.CompilerParams(
            needs_layout_passes=False, use_tc_tiling_on_sc=False
        ),
        scratch_types=[
            pltpu.VMEM((_IDX_PER_CHUNK,), jnp.int32),
            pltpu.VMEM((_IDX_PER_CHUNK,), jnp.int32),
            pltpu.VMEM((_IDX_PER_CHUNK, _NDIM), jnp.float32),
            pltpu.VMEM((_IDX_PER_CHUNK, _NDIM), jnp.float32),
            pltpu.VMEM((_HIST, _NDIM, _C + 1), jnp.float32),
            pltpu.VMEM((_HIST, _NDIM, _C + 1), jnp.float32),
            pltpu.SemaphoreType.DMA,
            pltpu.SemaphoreType.DMA,
            pltpu.SemaphoreType.DMA,
            pltpu.SemaphoreType.DMA,
        ],
    )
    def k2(idx_hbm, z_hbm, out_hbm, idx0, idx1, rows0, rows1,
           ob0, ob1, sg0, sg1, so0, so1):
        wid = lax.axis_index("s") * _NC + lax.axis_index("c")
        idx_base = wid * (_TRIALS_PER_W * _HIST)
        b_base = wid * _TRIALS_PER_W

        iota_d = jnp.arange(16, dtype=jnp.int32)

        def start(g, idx_v, rows_v, sg):
            i0 = idx_base + g * _IDX_PER_CHUNK
            pltpu.sync_copy(idx_hbm.at[pl.ds(i0, _IDX_PER_CHUNK)], idx_v)
            for j in range(_NG):
                pltpu.async_copy(
                    z_hbm.at[idx_v.at[pl.ds(j * _G, _G)]],
                    rows_v.at[pl.ds(j * _G, _G)],
                    sg,
                )

        def wait_gather(rows_v, sg):
            pltpu.make_async_copy(
                z_hbm.at[pl.ds(0, _IDX_PER_CHUNK)], rows_v, sg
            ).wait()

        def relayout(rows_v, out_v):
            iota_dlo = iota_d
            iota_dhi = iota_d + 16

            def bbody(b, carry):
                b_splat = jnp.full((16,), 0, jnp.int32) + b
                for h in range(_HIST):
                    r = b * _HIST + h
                    h_splat = jnp.full((16,), h, jnp.int32)
                    lo = rows_v[r, pl.ds(0, 16)]
                    hi = rows_v[r, pl.ds(16, 16)]
                    plsc.store_scatter(
                        out_v, [h_splat, iota_dlo, b_splat], lo
                    )
                    plsc.store_scatter(
                        out_v, [h_splat, iota_dhi, b_splat], hi
                    )
                return carry

            lax.fori_loop(0, _C, bbody, 0)

        def start_out(g, out_v, so):
            pltpu.async_copy(
                out_v.at[:, :, pl.ds(0, _C)],
                out_hbm.at[:, :, pl.ds(b_base + g * _C, _C)],
                so,
            )

        def wait_out(out_v, so):
            pltpu.make_async_copy(
                out_hbm.at[:, :, pl.ds(0, _C)], out_v.at[:, :, pl.ds(0, _C)], so
            ).wait()

        start(0, idx0, rows0, sg0)

        def pair(p, carry):
            g0 = 2 * p
            g1 = g0 + 1
            start(g1, idx1, rows1, sg1)
            wait_gather(rows0, sg0)

            @pl.when(p > 0)
            def _():
                wait_out(ob0, so0)

            relayout(rows0, ob0)
            start_out(g0, ob0, so0)

            @pl.when(p < _NCHUNK // 2 - 1)
            def _():
                start(g0 + 2, idx0, rows0, sg0)

            wait_gather(rows1, sg1)

            @pl.when(p > 0)
            def _():
                wait_out(ob1, so1)

            relayout(rows1, ob1)
            start_out(g1, ob1, so1)
            return carry

        lax.fori_loop(0, _NCHUNK // 2, pair, 0)
        wait_out(ob0, so0)
        wait_out(ob1, so1)

    return k2


_K1 = _build_k1()
_K2 = _build_k2()


@jax.jit
def kernel(inputs, z):
    zt = jnp.transpose(z)
    ztail = lax.slice(z, (_TAIL0, 0), (_NSTIM, _NDIM)).reshape(-1)
    z_lin = _K1(zt, ztail).reshape(_NSTIM, _NDIM)
    idx = jnp.reshape(inputs, (_BATCH * _HIST,))
    out_t = _K2(idx, z_lin)
    return jnp.transpose(out_t, (2, 1, 0))
